# Initial kernel scaffold; baseline (speedup 1.0000x reference)
import jax
import jax.numpy as jnp
from jax.experimental import pallas as pl


def kernel(hidden_states, weight, e_score_correction_bias):
    raise NotImplementedError("write your pallas kernel here")


# speedup vs baseline: 1.5016x; 1.5016x over previous
import functools

import jax
import jax.numpy as jnp
from jax.experimental import pallas as pl

_H = 2048
_E = 64
_TOP_K = 8
_N_GROUP = 8
_GROUP_SIZE = _E // _N_GROUP
_TOPK_GROUP = 4
_SCALE = 2.5
_NEG = float("-inf")


def _gate_kernel(x_ref, w_ref, b_ref, idx_ref, wgt_ref):
    x = x_ref[...]
    w = w_ref[...]
    logits = jnp.dot(x, w, preferred_element_type=jnp.float32)
    s = jax.nn.sigmoid(logits) + b_ref[...]
    t = s.shape[0]

    g = s.reshape(t, _N_GROUP, _GROUP_SIZE)
    lane3 = jax.lax.broadcasted_iota(jnp.int32, g.shape, 2)
    m1 = jnp.max(g, axis=-1, keepdims=True)
    first = jnp.min(jnp.where(g >= m1, lane3, _GROUP_SIZE), axis=-1, keepdims=True)
    m2 = jnp.max(jnp.where(lane3 == first, _NEG, g), axis=-1, keepdims=True)
    group_scores = (m1 + m2)[..., 0]

    grp_lane = jax.lax.broadcasted_iota(jnp.int32, (t, _N_GROUP), 1)
    lane = jax.lax.broadcasted_iota(jnp.int32, (t, _E), 1)
    grp_of_lane = lane // _GROUP_SIZE
    gs = group_scores
    smask = jnp.zeros((t, _E), jnp.bool_)
    for _ in range(_TOPK_GROUP):
        mx = jnp.max(gs, axis=-1, keepdims=True)
        sel = jnp.min(jnp.where(gs >= mx, grp_lane, _N_GROUP), axis=-1,
                      keepdims=True)
        smask = jnp.logical_or(smask, grp_of_lane == sel)
        gs = jnp.where(grp_lane == sel, _NEG, gs)

    tmp = jnp.where(smask, s, 0.0)
    idx_cols = []
    w_cols = []
    for _ in range(_TOP_K):
        mx = jnp.max(tmp, axis=-1, keepdims=True)
        sel = jnp.min(jnp.where(tmp >= mx, lane, _E), axis=-1, keepdims=True)
        hit = lane == sel
        wv = jnp.max(jnp.where(hit, s, _NEG), axis=-1, keepdims=True)
        idx_cols.append(sel)
        w_cols.append(wv)
        tmp = jnp.where(hit, _NEG, tmp)
    idx = jnp.concatenate(idx_cols, axis=1)
    wgt = jnp.concatenate(w_cols, axis=1)
    wgt = wgt / (jnp.sum(wgt, axis=-1, keepdims=True) + 1e-20) * _SCALE

    idx_ref[...] = idx
    wgt_ref[...] = wgt


@functools.partial(jax.jit, static_argnames=("interpret",))
def _gate(x2, weight, bias2, interpret=False):
    n = x2.shape[0]
    t = 512
    grid = (n // t,)
    return pl.pallas_call(
        _gate_kernel,
        grid=grid,
        in_specs=[
            pl.BlockSpec((t, _H), lambda i: (i, 0)),
            pl.BlockSpec((_H, _E), lambda i: (0, 0)),
            pl.BlockSpec((1, _E), lambda i: (0, 0)),
        ],
        out_specs=[
            pl.BlockSpec((t, _TOP_K), lambda i: (i, 0)),
            pl.BlockSpec((t, _TOP_K), lambda i: (i, 0)),
        ],
        out_shape=[
            jax.ShapeDtypeStruct((n, _TOP_K), jnp.int32),
            jax.ShapeDtypeStruct((n, _TOP_K), jnp.float32),
        ],
        interpret=interpret,
    )(x2, weight, bias2)


def kernel(hidden_states, weight, e_score_correction_bias):
    bsz, seq_len, h = hidden_states.shape
    x2 = hidden_states.reshape(bsz * seq_len, h)
    bias2 = e_score_correction_bias.reshape(1, _E)
    topk_idx, topk_weight = _gate(x2, weight, bias2)
    return (topk_idx, topk_weight)


# speedup vs baseline: 2.0239x; 1.3478x over previous
import functools

import jax
import jax.numpy as jnp
from jax.experimental import pallas as pl
from jax.experimental.pallas import tpu as pltpu

_H = 2048
_E = 64
_TOP_K = 8
_N_GROUP = 8
_GROUP_SIZE = _E // _N_GROUP
_TOPK_GROUP = 4
_SCALE = 2.5
_NEG = float("-inf")


def _xor_partner(x, lane, k):
    up = pltpu.roll(x, _E - k, 1)
    dn = pltpu.roll(x, k, 1)
    return jnp.where((lane & k) == 0, up, dn)


def _gate_kernel(x_ref, w_ref, b_ref, idx_ref, wgt_ref):
    x = x_ref[...]
    w = w_ref[...]
    logits = jnp.dot(x, w, preferred_element_type=jnp.float32)
    s = jax.nn.sigmoid(logits) + b_ref[...]
    t = s.shape[0]

    lane = jax.lax.broadcasted_iota(jnp.int32, (t, _E), 1)
    grp = lane // _GROUP_SIZE

    b1 = s
    b2 = jnp.full((t, _E), _NEG, jnp.float32)
    for k in (1, 2, 4):
        p1 = _xor_partner(b1, lane, k)
        p2 = _xor_partner(b2, lane, k)
        hi = jnp.maximum(b1, p1)
        lo = jnp.minimum(b1, p1)
        b2 = jnp.maximum(lo, jnp.maximum(b2, p2))
        b1 = hi
    gs = b1 + b2

    smask = jnp.zeros((t, _E), jnp.bool_)
    for _ in range(_TOPK_GROUP):
        mx = jnp.max(gs, axis=-1, keepdims=True)
        sel = jnp.min(jnp.where(gs >= mx, grp, _N_GROUP), axis=-1,
                      keepdims=True)
        hit = grp == sel
        smask = jnp.logical_or(smask, hit)
        gs = jnp.where(hit, _NEG, gs)

    tmp = jnp.where(smask, s, 0.0)
    acc_i = jnp.zeros((t, _E), jnp.int32)
    acc_w = jnp.zeros((t, _E), jnp.float32)
    for kk in range(_TOP_K):
        mx = jnp.max(tmp, axis=-1, keepdims=True)
        sel = jnp.min(jnp.where(tmp >= mx, lane, _E), axis=-1, keepdims=True)
        hit = lane == sel
        wv = jnp.max(jnp.where(hit, s, _NEG), axis=-1, keepdims=True)
        acc_i = jnp.where(lane == kk, sel, acc_i)
        acc_w = jnp.where(lane == kk, wv, acc_w)
        tmp = jnp.where(hit, _NEG, tmp)

    denom = jnp.sum(acc_w, axis=-1, keepdims=True) + 1e-20
    acc_w = acc_w * (_SCALE / denom)

    idx_ref[...] = acc_i[:, :_TOP_K]
    wgt_ref[...] = acc_w[:, :_TOP_K]


@functools.partial(jax.jit, static_argnames=("interpret",))
def _gate(x2, weight, bias2, interpret=False):
    n = x2.shape[0]
    t = 512
    grid = (n // t,)
    return pl.pallas_call(
        _gate_kernel,
        grid=grid,
        in_specs=[
            pl.BlockSpec((t, _H), lambda i: (i, 0)),
            pl.BlockSpec((_H, _E), lambda i: (0, 0)),
            pl.BlockSpec((1, _E), lambda i: (0, 0)),
        ],
        out_specs=[
            pl.BlockSpec((t, _TOP_K), lambda i: (i, 0)),
            pl.BlockSpec((t, _TOP_K), lambda i: (i, 0)),
        ],
        out_shape=[
            jax.ShapeDtypeStruct((n, _TOP_K), jnp.int32),
            jax.ShapeDtypeStruct((n, _TOP_K), jnp.float32),
        ],
        interpret=interpret,
    )(x2, weight, bias2)


def kernel(hidden_states, weight, e_score_correction_bias):
    bsz, seq_len, h = hidden_states.shape
    x2 = hidden_states.reshape(bsz * seq_len, h)
    bias2 = e_score_correction_bias.reshape(1, _E)
    topk_idx, topk_weight = _gate(x2, weight, bias2)
    return (topk_idx, topk_weight)


# speedup vs baseline: 2.6373x; 1.3031x over previous
import functools

import jax
import jax.numpy as jnp
from jax.experimental import pallas as pl
from jax.experimental.pallas import tpu as pltpu

_H = 2048
_E = 64
_TOP_K = 8
_N_GROUP = 8
_GROUP_SIZE = _E // _N_GROUP
_TOPK_GROUP = 4
_SCALE = 2.5
_NEG = float("-inf")


def _xor_partner(x, lane, k):
    up = pltpu.roll(x, _E - k, 1)
    dn = pltpu.roll(x, k, 1)
    return jnp.where((lane & k) == 0, up, dn)


def _gate_kernel(x_ref, w_ref, b_ref, idx_ref, wgt_ref):
    x = x_ref[...]
    w = w_ref[...]
    logits = jnp.dot(x, w, preferred_element_type=jnp.float32)
    s = jax.nn.sigmoid(logits) + b_ref[...]
    t = s.shape[0]

    lane = jax.lax.broadcasted_iota(jnp.int32, (t, _E), 1)
    lane_f = lane.astype(jnp.float32)
    grp_f = (lane // _GROUP_SIZE).astype(jnp.float32)

    p1 = _xor_partner(s, lane, 1)
    b1 = jnp.maximum(s, p1)
    b2 = jnp.minimum(s, p1)
    for k in (2, 4):
        p1 = _xor_partner(b1, lane, k)
        p2 = _xor_partner(b2, lane, k)
        lo = jnp.minimum(b1, p1)
        b1 = jnp.maximum(b1, p1)
        b2 = jnp.maximum(lo, jnp.maximum(b2, p2))
    gs = b1 + b2

    smask = jnp.zeros((t, _E), jnp.bool_)
    for _ in range(_TOPK_GROUP):
        mx = jnp.max(gs, axis=-1, keepdims=True)
        sel = jnp.min(jnp.where(gs >= mx, grp_f, 8.0), axis=-1,
                      keepdims=True)
        hit = grp_f == sel
        smask = jnp.logical_or(smask, hit)
        gs = jnp.where(hit, _NEG, gs)

    tmp = jnp.where(smask, s, 0.0)
    acc_i = jnp.zeros((t, _E), jnp.float32)
    acc_w = jnp.zeros((t, _E), jnp.float32)
    for kk in range(_TOP_K):
        mx = jnp.max(tmp, axis=-1, keepdims=True)
        sel = jnp.min(jnp.where(tmp >= mx, lane_f, 64.0), axis=-1,
                      keepdims=True)
        hit = lane_f == sel
        wv = jnp.max(jnp.where(hit, s, _NEG), axis=-1, keepdims=True)
        col = lane_f == float(kk)
        acc_i = jnp.where(col, sel, acc_i)
        acc_w = jnp.where(col, wv, acc_w)
        tmp = jnp.where(hit, _NEG, tmp)

    denom = jnp.sum(acc_w, axis=-1, keepdims=True) + 1e-20
    acc_w = acc_w * (_SCALE / denom)

    idx_ref[...] = acc_i.astype(jnp.int32)[:, :_TOP_K]
    wgt_ref[...] = acc_w[:, :_TOP_K]


@functools.partial(jax.jit, static_argnames=("interpret",))
def _gate(x2, weight, bias2, interpret=False):
    n = x2.shape[0]
    t = 512
    grid = (n // t,)
    return pl.pallas_call(
        _gate_kernel,
        grid=grid,
        in_specs=[
            pl.BlockSpec((t, _H), lambda i: (i, 0)),
            pl.BlockSpec((_H, _E), lambda i: (0, 0)),
            pl.BlockSpec((1, _E), lambda i: (0, 0)),
        ],
        out_specs=[
            pl.BlockSpec((t, _TOP_K), lambda i: (i, 0)),
            pl.BlockSpec((t, _TOP_K), lambda i: (i, 0)),
        ],
        out_shape=[
            jax.ShapeDtypeStruct((n, _TOP_K), jnp.int32),
            jax.ShapeDtypeStruct((n, _TOP_K), jnp.float32),
        ],
        interpret=interpret,
    )(x2, weight, bias2)


def kernel(hidden_states, weight, e_score_correction_bias):
    bsz, seq_len, h = hidden_states.shape
    x2 = hidden_states.reshape(bsz * seq_len, h)
    bias2 = e_score_correction_bias.reshape(1, _E)
    topk_idx, topk_weight = _gate(x2, weight, bias2)
    return (topk_idx, topk_weight)


# speedup vs baseline: 3.1450x; 1.1925x over previous
import functools

import jax
import jax.numpy as jnp
from jax.experimental import pallas as pl
from jax.experimental.pallas import tpu as pltpu

_H = 2048
_E = 64
_TOP_K = 8
_N_GROUP = 8
_GROUP_SIZE = _E // _N_GROUP
_TOPK_GROUP = 4
_SCALE = 2.5
_NEG = float("-inf")


def _shift_up(x, k):
    return pltpu.roll(x, _E - k, 1)


def _gate_kernel(x_ref, w_ref, b_ref, idx_ref, wgt_ref):
    x = x_ref[...]
    w = w_ref[...]
    logits = jnp.dot(x, w, preferred_element_type=jnp.float32)
    s = jax.nn.sigmoid(logits) + b_ref[...]
    t = s.shape[0]

    lane = jax.lax.broadcasted_iota(jnp.int32, (t, _E), 1)
    lane_f = lane.astype(jnp.float32)
    grp_f = (lane // _GROUP_SIZE).astype(jnp.float32)

    p1 = _shift_up(s, 1)
    b1 = jnp.maximum(s, p1)
    b2 = jnp.minimum(s, p1)
    for k in (2, 4):
        p1 = _shift_up(b1, k)
        p2 = _shift_up(b2, k)
        lo = jnp.minimum(b1, p1)
        b1 = jnp.maximum(b1, p1)
        b2 = jnp.maximum(lo, jnp.maximum(b2, p2))
    gs = jnp.where((lane & (_GROUP_SIZE - 1)) == 0, b1 + b2, _NEG)

    smask = jnp.zeros((t, _E), jnp.bool_)
    for _ in range(_TOPK_GROUP):
        mx = jnp.max(gs, axis=-1, keepdims=True)
        sel = jnp.min(jnp.where(gs >= mx, grp_f, 8.0), axis=-1,
                      keepdims=True)
        hit = grp_f == sel
        smask = jnp.logical_or(smask, hit)
        gs = jnp.where(hit, _NEG, gs)

    tmp = jnp.where(smask, s, 0.0)
    acc_i = jnp.zeros((t, _E), jnp.float32)
    acc_w = jnp.zeros((t, _E), jnp.float32)
    for kk in range(_TOP_K):
        mx = jnp.max(tmp, axis=-1, keepdims=True)
        sel = jnp.min(jnp.where(tmp >= mx, lane_f, 64.0), axis=-1,
                      keepdims=True)
        hit = lane_f == sel
        wv = jnp.max(jnp.where(hit, s, _NEG), axis=-1, keepdims=True)
        col = lane_f == float(kk)
        acc_i = jnp.where(col, sel, acc_i)
        acc_w = jnp.where(col, wv, acc_w)
        tmp = jnp.where(hit, _NEG, tmp)

    denom = jnp.sum(acc_w, axis=-1, keepdims=True) + 1e-20
    acc_w = acc_w * (_SCALE / denom)

    idx_ref[...] = acc_i.astype(jnp.int32)[:, :_TOP_K]
    wgt_ref[...] = acc_w[:, :_TOP_K]


@functools.partial(jax.jit, static_argnames=("interpret",))
def _gate(x2, weight, bias2, interpret=False):
    n = x2.shape[0]
    t = 1024
    grid = (n // t,)
    return pl.pallas_call(
        _gate_kernel,
        grid=grid,
        in_specs=[
            pl.BlockSpec((t, _H), lambda i: (i, 0)),
            pl.BlockSpec((_H, _E), lambda i: (0, 0)),
            pl.BlockSpec((1, _E), lambda i: (0, 0)),
        ],
        out_specs=[
            pl.BlockSpec((t, _TOP_K), lambda i: (i, 0)),
            pl.BlockSpec((t, _TOP_K), lambda i: (i, 0)),
        ],
        out_shape=[
            jax.ShapeDtypeStruct((n, _TOP_K), jnp.int32),
            jax.ShapeDtypeStruct((n, _TOP_K), jnp.float32),
        ],
        interpret=interpret,
    )(x2, weight, bias2)


def kernel(hidden_states, weight, e_score_correction_bias):
    bsz, seq_len, h = hidden_states.shape
    x2 = hidden_states.reshape(bsz * seq_len, h)
    bias2 = e_score_correction_bias.reshape(1, _E)
    topk_idx, topk_weight = _gate(x2, weight, bias2)
    return (topk_idx, topk_weight)


# speedup vs baseline: 7.7654x; 2.4691x over previous
import functools

import jax
import jax.numpy as jnp
from jax.experimental import pallas as pl
from jax.experimental.pallas import tpu as pltpu

_H = 2048
_E = 64
_TOP_K = 8
_N_GROUP = 8
_GROUP_SIZE = _E // _N_GROUP
_TOPK_GROUP = 4
_SCALE = 2.5
_NEG = float("-inf")


def _shift_up_rows(x, k):
    return pltpu.roll(x, _E - k, 0)


def _gate_kernel(x_ref, w_ref, b_ref, idx_ref, wgt_ref):
    x = x_ref[...]
    w = w_ref[...]
    logits = jnp.dot(x, w, preferred_element_type=jnp.float32)
    sT = jax.nn.sigmoid(logits.T) + b_ref[...]
    t = sT.shape[1]

    row = jax.lax.broadcasted_iota(jnp.int32, (_E, t), 0)
    row_f = row.astype(jnp.float32)
    grp_f = (row // _GROUP_SIZE).astype(jnp.float32)

    p1 = _shift_up_rows(sT, 1)
    b1 = jnp.maximum(sT, p1)
    b2 = jnp.minimum(sT, p1)
    for k in (2, 4):
        p1 = _shift_up_rows(b1, k)
        p2 = _shift_up_rows(b2, k)
        lo = jnp.minimum(b1, p1)
        b1 = jnp.maximum(b1, p1)
        b2 = jnp.maximum(lo, jnp.maximum(b2, p2))
    gs = jnp.where((row & (_GROUP_SIZE - 1)) == 0, b1 + b2, _NEG)

    smask = jnp.zeros((_E, t), jnp.bool_)
    for _ in range(_TOPK_GROUP):
        mx = jnp.max(gs, axis=0, keepdims=True)
        sel = jnp.min(jnp.where(gs >= mx, grp_f, 8.0), axis=0,
                      keepdims=True)
        hit = grp_f == sel
        smask = jnp.logical_or(smask, hit)
        gs = jnp.where(hit, _NEG, gs)

    tmp = jnp.where(smask, sT, 0.0)
    krow = jax.lax.broadcasted_iota(jnp.int32, (_TOP_K, t), 0)
    acc_i = jnp.zeros((_TOP_K, t), jnp.float32)
    acc_w = jnp.zeros((_TOP_K, t), jnp.float32)
    for kk in range(_TOP_K):
        mx = jnp.max(tmp, axis=0, keepdims=True)
        sel = jnp.min(jnp.where(tmp >= mx, row_f, 64.0), axis=0,
                      keepdims=True)
        hit = row_f == sel
        wv = jnp.max(jnp.where(hit, sT, _NEG), axis=0, keepdims=True)
        col = krow == kk
        acc_i = jnp.where(col, sel, acc_i)
        acc_w = jnp.where(col, wv, acc_w)
        tmp = jnp.where(hit, _NEG, tmp)

    denom = jnp.sum(acc_w, axis=0, keepdims=True) + 1e-20
    acc_w = acc_w * (_SCALE / denom)

    idx_ref[...] = acc_i.astype(jnp.int32)
    wgt_ref[...] = acc_w


@functools.partial(jax.jit, static_argnames=("interpret",))
def _gate(x2, weight, bias_col, interpret=False):
    n = x2.shape[0]
    t = 1024
    grid = (n // t,)
    return pl.pallas_call(
        _gate_kernel,
        grid=grid,
        in_specs=[
            pl.BlockSpec((t, _H), lambda i: (i, 0)),
            pl.BlockSpec((_H, _E), lambda i: (0, 0)),
            pl.BlockSpec((_E, 1), lambda i: (0, 0)),
        ],
        out_specs=[
            pl.BlockSpec((_TOP_K, t), lambda i: (0, i)),
            pl.BlockSpec((_TOP_K, t), lambda i: (0, i)),
        ],
        out_shape=[
            jax.ShapeDtypeStruct((_TOP_K, n), jnp.int32),
            jax.ShapeDtypeStruct((_TOP_K, n), jnp.float32),
        ],
        interpret=interpret,
    )(x2, weight, bias_col)


def kernel(hidden_states, weight, e_score_correction_bias):
    bsz, seq_len, h = hidden_states.shape
    x2 = hidden_states.reshape(bsz * seq_len, h)
    bias_col = e_score_correction_bias.reshape(_E, 1)
    idx_t, wgt_t = _gate(x2, weight, bias_col)
    return (idx_t.T, wgt_t.T)


# speedup vs baseline: 8.1306x; 1.0470x over previous
import functools

import jax
import jax.numpy as jnp
from jax.experimental import pallas as pl
from jax.experimental.pallas import tpu as pltpu

_H = 2048
_E = 64
_TOP_K = 8
_N_GROUP = 8
_GROUP_SIZE = _E // _N_GROUP
_TOPK_GROUP = 4
_SCALE = 2.5
_NEG = float("-inf")


def _shift_up_rows(x, k):
    return pltpu.roll(x, _E - k, 0)


def _gate_kernel(x_ref, w_ref, b_ref, idx_ref, wgt_ref):
    x = x_ref[...]
    w = w_ref[...]
    logits = jnp.dot(x, w, preferred_element_type=jnp.float32)
    sT = jax.nn.sigmoid(logits.T) + b_ref[...]
    t = sT.shape[1]

    row = jax.lax.broadcasted_iota(jnp.int32, (_E, t), 0)
    row_f = row.astype(jnp.float32)
    grp_f = (row // _GROUP_SIZE).astype(jnp.float32)

    p1 = _shift_up_rows(sT, 1)
    b1 = jnp.maximum(sT, p1)
    b2 = jnp.minimum(sT, p1)
    for k in (2, 4):
        p1 = _shift_up_rows(b1, k)
        p2 = _shift_up_rows(b2, k)
        lo = jnp.minimum(b1, p1)
        b1 = jnp.maximum(b1, p1)
        b2 = jnp.maximum(lo, jnp.maximum(b2, p2))
    gs = jnp.where((row & (_GROUP_SIZE - 1)) == 0, b1 + b2, _NEG)

    smask = jnp.zeros((_E, t), jnp.bool_)
    for _ in range(_TOPK_GROUP):
        mx = jnp.max(gs, axis=0, keepdims=True)
        sel = jnp.min(jnp.where(gs >= mx, grp_f, 8.0), axis=0,
                      keepdims=True)
        hit = grp_f == sel
        smask = jnp.logical_or(smask, hit)
        gs = jnp.where(hit, _NEG, gs)

    tmp = jnp.where(smask, sT, 0.0)
    krow = jax.lax.broadcasted_iota(jnp.int32, (_TOP_K, t), 0)
    acc_i = jnp.zeros((_TOP_K, t), jnp.float32)
    acc_w = jnp.zeros((_TOP_K, t), jnp.float32)
    for kk in range(_TOP_K):
        mx = jnp.max(tmp, axis=0, keepdims=True)
        sel = jnp.min(jnp.where(tmp >= mx, row_f, 64.0), axis=0,
                      keepdims=True)
        hit = row_f == sel
        wv = jnp.max(jnp.where(hit, sT, _NEG), axis=0, keepdims=True)
        col = krow == kk
        acc_i = jnp.where(col, sel, acc_i)
        acc_w = jnp.where(col, wv, acc_w)
        tmp = jnp.where(hit, _NEG, tmp)

    denom = jnp.sum(acc_w, axis=0, keepdims=True) + 1e-20
    acc_w = acc_w * (_SCALE / denom)

    idx_ref[...] = acc_i.astype(jnp.int32)
    wgt_ref[...] = acc_w


@functools.partial(jax.jit, static_argnames=("interpret",))
def _gate(x2, weight, bias_col, interpret=False):
    n = x2.shape[0]
    t = 2048
    grid = (n // t,)
    return pl.pallas_call(
        _gate_kernel,
        grid=grid,
        in_specs=[
            pl.BlockSpec((t, _H), lambda i: (i, 0)),
            pl.BlockSpec((_H, _E), lambda i: (0, 0)),
            pl.BlockSpec((_E, 1), lambda i: (0, 0)),
        ],
        out_specs=[
            pl.BlockSpec((_TOP_K, t), lambda i: (0, i)),
            pl.BlockSpec((_TOP_K, t), lambda i: (0, i)),
        ],
        out_shape=[
            jax.ShapeDtypeStruct((_TOP_K, n), jnp.int32),
            jax.ShapeDtypeStruct((_TOP_K, n), jnp.float32),
        ],
        interpret=interpret,
    )(x2, weight, bias_col)


def kernel(hidden_states, weight, e_score_correction_bias):
    bsz, seq_len, h = hidden_states.shape
    x2 = hidden_states.reshape(bsz * seq_len, h)
    bias_col = e_score_correction_bias.reshape(_E, 1)
    idx_t, wgt_t = _gate(x2, weight, bias_col)
    return (idx_t.T, wgt_t.T)
